# Initial kernel scaffold; baseline (speedup 1.0000x reference)
import jax
import jax.numpy as jnp
from jax.experimental import pallas as pl


def kernel(voxel_tokens, non_empty_mask, g1, b1, g2, b2, Wq, bq, Wk, bk, Wv, bv, Wo, bo, W1, bm1, W2, bm2):
    raise NotImplementedError("write your pallas kernel here")


# speedup vs baseline: 10.8147x; 10.8147x over previous
import functools

import jax
import jax.numpy as jnp
from jax import lax
from jax.experimental import pallas as pl
from jax.experimental.pallas import tpu as pltpu
from jax.experimental.pallas import tpu_sc as plsc

_R = 16
_V = _R ** 3
_B = 2
_C = 384
_H = 8
_DH = _C // _H
_K = 16
_TK = 8
_MLP = 1536
_TILE = 256
_NTILE = _V // _TILE
_BIG_D2 = 676


def _qkv_knn_body(tok_ref, mask_ref, g1_ref, b1_ref, wq_ref, bq_ref,
                  wk_ref, bk_ref, wv_ref, bv_ref,
                  q_ref, k_ref, v_ref, idx_ref, val_ref):
    b = pl.program_id(0)
    i = pl.program_id(1)

    x = tok_ref[0]
    mu = jnp.mean(x, axis=1, keepdims=True)
    var = jnp.mean((x - mu) ** 2, axis=1, keepdims=True)
    xn = (x - mu) * lax.rsqrt(var + 1e-5)
    xn = xn * g1_ref[0] + b1_ref[0]

    q_ref[0] = jnp.dot(xn, wq_ref[...], preferred_element_type=jnp.float32) + bq_ref[0]
    k_ref[0] = jnp.dot(xn, wk_ref[...], preferred_element_type=jnp.float32) + bk_ref[0]
    v_ref[0] = jnp.dot(xn, wv_ref[...], preferred_element_type=jnp.float32) + bv_ref[0]

    rr = lax.broadcasted_iota(jnp.int32, (_TILE, _V), 0) + i * _TILE
    cc = lax.broadcasted_iota(jnp.int32, (_TILE, _V), 1)
    dx = (rr >> 8) - (cc >> 8)
    dy = ((rr >> 4) & 15) - ((cc >> 4) & 15)
    dz = (rr & 15) - (cc & 15)
    dxf = dx.astype(jnp.float32)
    dyf = dy.astype(jnp.float32)
    dzf = dz.astype(jnp.float32)
    d2 = dxf * dxf + dyf * dyf + dzf * dzf
    mrow = mask_ref[0]
    d2 = jnp.where(mrow != 0, d2, float(_BIG_D2))
    keys = d2 * float(_V) + cc.astype(jnp.float32)

    valid_cut = float(_BIG_D2 * _V)
    idx_cols = []
    val_cols = []
    for _ in range(_K):
        m = jnp.min(keys, axis=1, keepdims=True)
        mi = m.astype(jnp.int32)
        idx_cols.append((mi & (_V - 1)) + b * _V)
        val_cols.append((m < valid_cut).astype(jnp.int32))
        keys = jnp.where(keys == m, 3e38, keys)
    idx_ref[0] = jnp.concatenate(idx_cols, axis=1)
    val_ref[0] = jnp.concatenate(val_cols, axis=1)


def _run_qkv_knn(tok, mask_i32, g1, b1, wq, bq, wk, bk, wv, bv):
    full = lambda s: pl.BlockSpec(s, lambda b, i: (0,) * len(s))
    grid = (_B, _NTILE)
    return pl.pallas_call(
        _qkv_knn_body,
        grid=grid,
        in_specs=[
            pl.BlockSpec((1, _TILE, _C), lambda b, i: (b, i, 0)),
            pl.BlockSpec((1, 1, _V), lambda b, i: (b, 0, 0)),
            full((1, _C)), full((1, _C)),
            full((_C, _C)), full((1, _C)),
            full((_C, _C)), full((1, _C)),
            full((_C, _C)), full((1, _C)),
        ],
        out_specs=[
            pl.BlockSpec((1, _TILE, _C), lambda b, i: (b, i, 0)),
            pl.BlockSpec((1, _TILE, _C), lambda b, i: (b, i, 0)),
            pl.BlockSpec((1, _TILE, _C), lambda b, i: (b, i, 0)),
            pl.BlockSpec((1, _TILE, _K), lambda b, i: (b, i, 0)),
            pl.BlockSpec((1, _TILE, _K), lambda b, i: (b, i, 0)),
        ],
        out_shape=[
            jax.ShapeDtypeStruct((_B, _V, _C), jnp.float32),
            jax.ShapeDtypeStruct((_B, _V, _C), jnp.float32),
            jax.ShapeDtypeStruct((_B, _V, _C), jnp.float32),
            jax.ShapeDtypeStruct((_B, _V, _K), jnp.int32),
            jax.ShapeDtypeStruct((_B, _V, _K), jnp.int32),
        ],
    )(tok, mask_i32, g1, b1, wq, bq, wk, bk, wv, bv)


_NIDX = _B * _V * _K
_GCH = 128


def _sc_gather(ktab, vtab, idx):
    info = plsc.get_sparse_core_info()
    nw = info.num_cores * info.num_subcores
    per_w = _NIDX // nw
    n_ch = per_w // _GCH
    mesh = plsc.VectorSubcoreMesh(core_axis_name="c", subcore_axis_name="s")

    @functools.partial(
        pl.kernel,
        mesh=mesh,
        out_type=[
            jax.ShapeDtypeStruct((_NIDX, _C), jnp.float32),
            jax.ShapeDtypeStruct((_NIDX, _C), jnp.float32),
        ],
        scratch_types=[
            pltpu.VMEM((_GCH,), jnp.int32),
            pltpu.VMEM((_GCH, _C), jnp.float32),
            pltpu.VMEM((_GCH, _C), jnp.float32),
            pltpu.SemaphoreType.DMA,
            pltpu.SemaphoreType.DMA,
        ],
    )
    def gath(ktab_hbm, vtab_hbm, idx_hbm, outk_hbm, outv_hbm,
             idx_v, kbuf, vbuf, sem_k, sem_v):
        wid = lax.axis_index("s") * info.num_cores + lax.axis_index("c")
        base = wid * per_w

        def step(g, carry):
            off = base + g * _GCH
            pltpu.sync_copy(idx_hbm.at[pl.ds(off, _GCH)], idx_v)
            ck = pltpu.async_copy(ktab_hbm.at[idx_v], kbuf, sem_k)
            cv = pltpu.async_copy(vtab_hbm.at[idx_v], vbuf, sem_v)
            ck.wait()
            cv.wait()
            pltpu.sync_copy(kbuf, outk_hbm.at[pl.ds(off, _GCH)])
            pltpu.sync_copy(vbuf, outv_hbm.at[pl.ds(off, _GCH)])
            return carry

        lax.fori_loop(0, n_ch, step, 0)

    return gath(ktab, vtab, idx)


def _attn_mlp_body(tok_ref, q_ref, knb_ref, vnb_ref, val_ref, maskf_ref,
                   g2_ref, b2_ref, wo_ref, bo_ref,
                   w1_ref, bm1_ref, w2_ref, bm2_ref, out_ref):
    q = q_ref[0].reshape(_TILE, 1, _H, _DH)
    knb = knb_ref[...].reshape(_TILE, _K, _H, _DH)
    scores = jnp.sum(q * knb, axis=-1) * (1.0 / (_DH ** 0.5))
    valid3 = val_ref[0].reshape(_TILE, _K, 1) != 0
    scores = jnp.where(valid3, scores, -1e30)

    rem = scores
    cum = jnp.zeros((_TILE, 1, _H), jnp.float32)
    thresh = jnp.full((_TILE, 1, _H), -3e38, jnp.float32)
    for _ in range(_TK):
        mt = jnp.max(rem, axis=1, keepdims=True)
        eq = rem == mt
        cnt = jnp.sum(eq.astype(jnp.float32), axis=1, keepdims=True)
        thresh = jnp.where(cum < float(_TK), mt, thresh)
        cum = cum + cnt
        rem = jnp.where(eq, -3e38, rem)

    s2 = jnp.where(scores >= thresh, scores, -1e30)
    smax = jnp.max(s2, axis=1, keepdims=True)
    e = jnp.exp(s2 - smax)
    e = jnp.where(s2 > -1e29, e, 0.0)
    attn = e / (jnp.sum(e, axis=1, keepdims=True) + 1e-9)

    vnb = vnb_ref[...].reshape(_TILE, _K, _H, _DH)
    out = jnp.sum(attn[..., None] * vnb, axis=1)
    out = out.reshape(_TILE, _C)
    proj = jnp.dot(out, wo_ref[...], preferred_element_type=jnp.float32) + bo_ref[0]
    proj = proj * maskf_ref[0][:, :1]
    x1 = proj * 0.5 + tok_ref[0]

    mu = jnp.mean(x1, axis=1, keepdims=True)
    var = jnp.mean((x1 - mu) ** 2, axis=1, keepdims=True)
    xn = (x1 - mu) * lax.rsqrt(var + 1e-5) * g2_ref[0] + b2_ref[0]
    h = jnp.dot(xn, w1_ref[...], preferred_element_type=jnp.float32) + bm1_ref[0]
    h = jax.nn.gelu(h)
    mlp = jnp.dot(h, w2_ref[...], preferred_element_type=jnp.float32) + bm2_ref[0]
    out_ref[0] = mlp * 0.5 + x1


def _run_attn_mlp(tok, q, knb, vnb, val, maskf, g2, b2, wo, bo, w1, bm1, w2, bm2):
    full = lambda s: pl.BlockSpec(s, lambda b, i: (0,) * len(s))
    grid = (_B, _NTILE)
    return pl.pallas_call(
        _attn_mlp_body,
        grid=grid,
        in_specs=[
            pl.BlockSpec((1, _TILE, _C), lambda b, i: (b, i, 0)),
            pl.BlockSpec((1, _TILE, _C), lambda b, i: (b, i, 0)),
            pl.BlockSpec((_TILE * _K, _C), lambda b, i: (b * _NTILE + i, 0)),
            pl.BlockSpec((_TILE * _K, _C), lambda b, i: (b * _NTILE + i, 0)),
            pl.BlockSpec((1, _TILE, _K), lambda b, i: (b, i, 0)),
            pl.BlockSpec((1, _TILE, 128), lambda b, i: (b, i, 0)),
            full((1, _C)), full((1, _C)),
            full((_C, _C)), full((1, _C)),
            full((_C, _MLP)), full((1, _MLP)),
            full((_MLP, _C)), full((1, _C)),
        ],
        out_specs=pl.BlockSpec((1, _TILE, _C), lambda b, i: (b, i, 0)),
        out_shape=jax.ShapeDtypeStruct((_B, _V, _C), jnp.float32),
    )(tok, q, knb, vnb, val, maskf, g2, b2, wo, bo, w1, bm1, w2, bm2)


def kernel(voxel_tokens, non_empty_mask, g1, b1, g2, b2, Wq, bq, Wk, bk,
           Wv, bv, Wo, bo, W1, bm1, W2, bm2):
    mask_i32 = non_empty_mask.astype(jnp.int32).reshape(_B, 1, _V)
    r2 = lambda a: a.reshape(1, -1)

    q, k, v, idx, val = _run_qkv_knn(
        voxel_tokens, mask_i32, r2(g1), r2(b1),
        Wq, r2(bq), Wk, r2(bk), Wv, r2(bv))

    knb, vnb = _sc_gather(
        k.reshape(_B * _V, _C), v.reshape(_B * _V, _C),
        idx.reshape(_B * _V * _K))

    maskf = jnp.broadcast_to(
        non_empty_mask.astype(jnp.float32).reshape(_B, _V, 1), (_B, _V, 128))
    return _run_attn_mlp(
        voxel_tokens, q, knb, vnb, val, maskf, r2(g2), r2(b2),
        Wo, r2(bo), W1, r2(bm1), W2, r2(bm2))


# speedup vs baseline: 22.8088x; 2.1091x over previous
import functools

import jax
import jax.numpy as jnp
from jax import lax
from jax.experimental import pallas as pl
from jax.experimental.pallas import tpu as pltpu
from jax.experimental.pallas import tpu_sc as plsc

_R = 16
_V = _R ** 3
_B = 2
_C = 384
_H = 8
_DH = _C // _H
_K = 16
_TK = 8
_MLP = 1536
_TILE = 256
_NTILE = _V // _TILE
_BIG_D2 = 676


def _qkv_knn_body(tok_ref, mask_ref, g1_ref, b1_ref, wq_ref, bq_ref,
                  wk_ref, bk_ref, wv_ref, bv_ref,
                  q_ref, k_ref, v_ref, idx_ref, val_ref):
    b = pl.program_id(0)
    i = pl.program_id(1)

    x = tok_ref[0]
    mu = jnp.mean(x, axis=1, keepdims=True)
    var = jnp.mean((x - mu) ** 2, axis=1, keepdims=True)
    xn = (x - mu) * lax.rsqrt(var + 1e-5)
    xn = xn * g1_ref[0] + b1_ref[0]

    q_ref[0] = jnp.dot(xn, wq_ref[...], preferred_element_type=jnp.float32) + bq_ref[0]
    k_ref[0] = jnp.dot(xn, wk_ref[...], preferred_element_type=jnp.float32) + bk_ref[0]
    v_ref[0] = jnp.dot(xn, wv_ref[...], preferred_element_type=jnp.float32) + bv_ref[0]

    rr = lax.broadcasted_iota(jnp.int32, (_TILE, _V), 0) + i * _TILE
    cc = lax.broadcasted_iota(jnp.int32, (_TILE, _V), 1)
    dx = (rr >> 8) - (cc >> 8)
    dy = ((rr >> 4) & 15) - ((cc >> 4) & 15)
    dz = (rr & 15) - (cc & 15)
    dxf = dx.astype(jnp.float32)
    dyf = dy.astype(jnp.float32)
    dzf = dz.astype(jnp.float32)
    d2 = dxf * dxf + dyf * dyf + dzf * dzf
    mrow = mask_ref[0]
    d2 = jnp.where(mrow != 0, d2, float(_BIG_D2))
    keys = d2 * float(_V) + cc.astype(jnp.float32)

    valid_cut = float(_BIG_D2 * _V)
    idx_cols = []
    val_cols = []
    for _ in range(_K):
        m = jnp.min(keys, axis=1, keepdims=True)
        mi = m.astype(jnp.int32)
        idx_cols.append((mi & (_V - 1)) + b * _V)
        val_cols.append((m < valid_cut).astype(jnp.int32))
        keys = jnp.where(keys == m, 3e38, keys)
    idx_ref[...] = jnp.concatenate(idx_cols, axis=1).T
    val_ref[...] = jnp.concatenate(val_cols, axis=1).T


def _run_qkv_knn(tok, mask_i32, g1, b1, wq, bq, wk, bk, wv, bv):
    full = lambda s: pl.BlockSpec(s, lambda b, i: (0,) * len(s))
    grid = (_B, _NTILE)
    return pl.pallas_call(
        _qkv_knn_body,
        grid=grid,
        in_specs=[
            pl.BlockSpec((1, _TILE, _C), lambda b, i: (b, i, 0)),
            pl.BlockSpec((1, 1, _V), lambda b, i: (b, 0, 0)),
            full((1, _C)), full((1, _C)),
            full((_C, _C)), full((1, _C)),
            full((_C, _C)), full((1, _C)),
            full((_C, _C)), full((1, _C)),
        ],
        out_specs=[
            pl.BlockSpec((1, _TILE, _C), lambda b, i: (b, i, 0)),
            pl.BlockSpec((1, _TILE, _C), lambda b, i: (b, i, 0)),
            pl.BlockSpec((1, _TILE, _C), lambda b, i: (b, i, 0)),
            pl.BlockSpec((_K, _TILE), lambda b, i: (0, b * _NTILE + i)),
            pl.BlockSpec((_K, _TILE), lambda b, i: (0, b * _NTILE + i)),
        ],
        out_shape=[
            jax.ShapeDtypeStruct((_B, _V, _C), jnp.float32),
            jax.ShapeDtypeStruct((_B, _V, _C), jnp.float32),
            jax.ShapeDtypeStruct((_B, _V, _C), jnp.float32),
            jax.ShapeDtypeStruct((_K, _B * _V), jnp.int32),
            jax.ShapeDtypeStruct((_K, _B * _V), jnp.int32),
        ],
    )(tok, mask_i32, g1, b1, wq, bq, wk, bk, wv, bv)


_NIDX = _B * _V * _K
_GCH = 128


def _sc_gather(ktab, vtab, idx):
    info = plsc.get_sparse_core_info()
    nw = info.num_cores * info.num_subcores
    per_w = _NIDX // nw
    n_ch = per_w // _GCH
    mesh = plsc.VectorSubcoreMesh(core_axis_name="c", subcore_axis_name="s")

    @functools.partial(
        pl.kernel,
        mesh=mesh,
        out_type=[
            jax.ShapeDtypeStruct((_NIDX, _C), jnp.float32),
            jax.ShapeDtypeStruct((_NIDX, _C), jnp.float32),
        ],
        scratch_types=[
            pltpu.VMEM((_GCH,), jnp.int32),
            pltpu.VMEM((_GCH, _C), jnp.float32),
            pltpu.VMEM((_GCH, _C), jnp.float32),
            pltpu.SemaphoreType.DMA,
            pltpu.SemaphoreType.DMA,
        ],
    )
    def gath(ktab_hbm, vtab_hbm, idx_hbm, outk_hbm, outv_hbm,
             idx_v, kbuf, vbuf, sem_k, sem_v):
        wid = lax.axis_index("s") * info.num_cores + lax.axis_index("c")
        base = wid * per_w

        def step(g, carry):
            off = base + g * _GCH
            pltpu.sync_copy(idx_hbm.at[pl.ds(off, _GCH)], idx_v)
            ck = pltpu.async_copy(ktab_hbm.at[idx_v], kbuf, sem_k)
            cv = pltpu.async_copy(vtab_hbm.at[idx_v], vbuf, sem_v)
            ck.wait()
            cv.wait()
            pltpu.sync_copy(kbuf, outk_hbm.at[pl.ds(off, _GCH)])
            pltpu.sync_copy(vbuf, outv_hbm.at[pl.ds(off, _GCH)])
            return carry

        lax.fori_loop(0, n_ch, step, 0)

    return gath(ktab, vtab, idx)


def _attn_mlp_body(tok_ref, q_ref, knb_ref, vnb_ref, val_ref, maskf_ref,
                   g2_ref, b2_ref, wo_ref, bo_ref,
                   w1_ref, bm1_ref, w2_ref, bm2_ref, out_ref):
    dd = lax.broadcasted_iota(jnp.int32, (_C, _H), 0) // _DH
    hh = lax.broadcasted_iota(jnp.int32, (_C, _H), 1)
    seg = jnp.where(dd == hh, 1.0 / (_DH ** 0.5), 0.0)
    dd2 = lax.broadcasted_iota(jnp.int32, (_H, _C), 1) // _DH
    hh2 = lax.broadcasted_iota(jnp.int32, (_H, _C), 0)
    segT = jnp.where(dd2 == hh2, 1.0, 0.0)

    q = q_ref[0]
    s_list = []
    for k in range(_K):
        prod_k = q * knb_ref[k]
        s_k = jnp.dot(prod_k, seg, preferred_element_type=jnp.float32)
        s_kT = s_k.T
        val_k = val_ref[pl.ds(k, 1), :]
        s_list.append(jnp.where(val_k != 0, s_kT, -1e30))

    rem = list(s_list)
    cum = jnp.zeros((_H, _TILE), jnp.float32)
    thresh = jnp.full((_H, _TILE), -3e38, jnp.float32)
    for _ in range(_TK):
        mt = rem[0]
        for r in rem[1:]:
            mt = jnp.maximum(mt, r)
        eqs = [r == mt for r in rem]
        cnt = eqs[0].astype(jnp.float32)
        for e_ in eqs[1:]:
            cnt = cnt + e_.astype(jnp.float32)
        thresh = jnp.where(cum < float(_TK), mt, thresh)
        cum = cum + cnt
        rem = [jnp.where(e_, -3e38, r) for e_, r in zip(eqs, rem)]

    s2_list = [jnp.where(s >= thresh, s, -1e30) for s in s_list]
    smax = s2_list[0]
    for s in s2_list[1:]:
        smax = jnp.maximum(smax, s)
    e_list = []
    esum = jnp.full((_H, _TILE), 1e-9, jnp.float32)
    for s in s2_list:
        e_ = jnp.exp(s - smax)
        e_ = jnp.where(s > -1e29, e_, 0.0)
        e_list.append(e_)
        esum = esum + e_
    inv = 1.0 / esum

    out = jnp.zeros((_TILE, _C), jnp.float32)
    for k in range(_K):
        attn_k = e_list[k] * inv
        af = lax.dot_general(attn_k, segT,
                             (((0,), (0,)), ((), ())),
                             preferred_element_type=jnp.float32)
        out = out + af * vnb_ref[k]
    proj = jnp.dot(out, wo_ref[...], preferred_element_type=jnp.float32) + bo_ref[0]
    proj = proj * maskf_ref[0]
    x1 = proj * 0.5 + tok_ref[0]

    mu = jnp.mean(x1, axis=1, keepdims=True)
    var = jnp.mean((x1 - mu) ** 2, axis=1, keepdims=True)
    xn = (x1 - mu) * lax.rsqrt(var + 1e-5) * g2_ref[0] + b2_ref[0]
    h = jnp.dot(xn, w1_ref[...], preferred_element_type=jnp.float32) + bm1_ref[0]
    h = jax.nn.gelu(h)
    mlp = jnp.dot(h, w2_ref[...], preferred_element_type=jnp.float32) + bm2_ref[0]
    out_ref[0] = mlp * 0.5 + x1


def _run_attn_mlp(tok, q, knb, vnb, val, maskf, g2, b2, wo, bo, w1, bm1, w2, bm2):
    full = lambda s: pl.BlockSpec(s, lambda b, i: (0,) * len(s))
    grid = (_B, _NTILE)
    return pl.pallas_call(
        _attn_mlp_body,
        grid=grid,
        in_specs=[
            pl.BlockSpec((1, _TILE, _C), lambda b, i: (b, i, 0)),
            pl.BlockSpec((1, _TILE, _C), lambda b, i: (b, i, 0)),
            pl.BlockSpec((_K, _TILE, _C), lambda b, i: (0, b * _NTILE + i, 0)),
            pl.BlockSpec((_K, _TILE, _C), lambda b, i: (0, b * _NTILE + i, 0)),
            pl.BlockSpec((_K, _TILE), lambda b, i: (0, b * _NTILE + i)),
            pl.BlockSpec((1, _TILE, 1), lambda b, i: (b, i, 0)),
            full((1, _C)), full((1, _C)),
            full((_C, _C)), full((1, _C)),
            full((_C, _MLP)), full((1, _MLP)),
            full((_MLP, _C)), full((1, _C)),
        ],
        out_specs=pl.BlockSpec((1, _TILE, _C), lambda b, i: (b, i, 0)),
        out_shape=jax.ShapeDtypeStruct((_B, _V, _C), jnp.float32),
    )(tok, q, knb, vnb, val, maskf, g2, b2, wo, bo, w1, bm1, w2, bm2)


def kernel(voxel_tokens, non_empty_mask, g1, b1, g2, b2, Wq, bq, Wk, bk,
           Wv, bv, Wo, bo, W1, bm1, W2, bm2):
    mask_i32 = non_empty_mask.astype(jnp.int32).reshape(_B, 1, _V)
    r2 = lambda a: a.reshape(1, -1)

    q, k, v, idx, val = _run_qkv_knn(
        voxel_tokens, mask_i32, r2(g1), r2(b1),
        Wq, r2(bq), Wk, r2(bk), Wv, r2(bv))

    knb, vnb = _sc_gather(
        k.reshape(_B * _V, _C), v.reshape(_B * _V, _C),
        idx.reshape(_K * _B * _V))

    maskf = non_empty_mask.astype(jnp.float32).reshape(_B, _V, 1)
    return _run_attn_mlp(
        voxel_tokens, q, knb.reshape(_K, _B * _V, _C),
        vnb.reshape(_K, _B * _V, _C), val, maskf, r2(g2), r2(b2),
        Wo, r2(bo), W1, r2(bm1), W2, r2(bm2))


# speedup vs baseline: 27.0550x; 1.1862x over previous
import functools

import jax
import jax.numpy as jnp
from jax import lax
from jax.experimental import pallas as pl
from jax.experimental.pallas import tpu as pltpu
from jax.experimental.pallas import tpu_sc as plsc

_R = 16
_V = _R ** 3
_B = 2
_C = 384
_H = 8
_DH = _C // _H
_K = 16
_TK = 8
_MLP = 1536
_TILE = 256
_NTILE = _V // _TILE
_BIG_D2 = 676


def _pack_bf16(lo, hi):
    def rnd(x):
        u = lax.bitcast_convert_type(x, jnp.int32)
        return lax.shift_right_logical(
            u + 0x7FFF + (lax.shift_right_logical(u, 16) & 1), 16)
    return rnd(lo) | (rnd(hi) << 16)


def _unpack_bf16(words):
    lo = lax.bitcast_convert_type(words << 16, jnp.float32)
    hi = lax.bitcast_convert_type(words & jnp.int32(-65536), jnp.float32)
    return lo, hi


def _qkv_knn_body(tok_ref, mask_ref, g1_ref, b1_ref, wq_ref, bq_ref,
                  wk_ref, bk_ref, wv_ref, bv_ref,
                  q_ref, kv_ref, idx_ref, val_ref):
    b = pl.program_id(0)
    i = pl.program_id(1)

    x = tok_ref[0]
    mu = jnp.mean(x, axis=1, keepdims=True)
    var = jnp.mean((x - mu) ** 2, axis=1, keepdims=True)
    xn = (x - mu) * lax.rsqrt(var + 1e-5)
    xn = xn * g1_ref[0] + b1_ref[0]

    q = jnp.dot(xn, wq_ref[...], preferred_element_type=jnp.float32) + bq_ref[0]
    k = jnp.dot(xn, wk_ref[...], preferred_element_type=jnp.float32) + bk_ref[0]
    v = jnp.dot(xn, wv_ref[...], preferred_element_type=jnp.float32) + bv_ref[0]
    hc = _C // 2
    q_ref[0] = _pack_bf16(q[:, :hc], q[:, hc:])
    kv_ref[0] = jnp.concatenate(
        [_pack_bf16(k[:, :hc], k[:, hc:]), _pack_bf16(v[:, :hc], v[:, hc:])],
        axis=1)

    rr = lax.broadcasted_iota(jnp.int32, (_TILE, _V), 0) + i * _TILE
    cc = lax.broadcasted_iota(jnp.int32, (_TILE, _V), 1)
    dx = (rr >> 8) - (cc >> 8)
    dy = ((rr >> 4) & 15) - ((cc >> 4) & 15)
    dz = (rr & 15) - (cc & 15)
    dxf = dx.astype(jnp.float32)
    dyf = dy.astype(jnp.float32)
    dzf = dz.astype(jnp.float32)
    d2 = dxf * dxf + dyf * dyf + dzf * dzf
    mrow = mask_ref[0]
    d2 = jnp.where(mrow != 0, d2, float(_BIG_D2))
    keys = d2 * float(_V) + cc.astype(jnp.float32)

    valid_cut = float(_BIG_D2 * _V)
    idx_cols = []
    val_cols = []
    for _ in range(_K):
        m = jnp.min(keys, axis=1, keepdims=True)
        mi = m.astype(jnp.int32)
        idx_cols.append((mi & (_V - 1)) + b * _V)
        val_cols.append((m < valid_cut).astype(jnp.int32))
        keys = jnp.where(keys == m, 3e38, keys)
    idx_ref[...] = jnp.concatenate(idx_cols, axis=1).T
    val_ref[...] = jnp.concatenate(val_cols, axis=1).T


def _run_qkv_knn(tok, mask_i32, g1, b1, wq, bq, wk, bk, wv, bv):
    full = lambda s: pl.BlockSpec(s, lambda b, i: (0,) * len(s))
    grid = (_B, _NTILE)
    return pl.pallas_call(
        _qkv_knn_body,
        grid=grid,
        in_specs=[
            pl.BlockSpec((1, _TILE, _C), lambda b, i: (b, i, 0)),
            pl.BlockSpec((1, 1, _V), lambda b, i: (b, 0, 0)),
            full((1, _C)), full((1, _C)),
            full((_C, _C)), full((1, _C)),
            full((_C, _C)), full((1, _C)),
            full((_C, _C)), full((1, _C)),
        ],
        out_specs=[
            pl.BlockSpec((1, _TILE, _C // 2), lambda b, i: (b, i, 0)),
            pl.BlockSpec((1, _TILE, _C), lambda b, i: (b, i, 0)),
            pl.BlockSpec((_K, _TILE), lambda b, i: (0, b * _NTILE + i)),
            pl.BlockSpec((_K, _TILE), lambda b, i: (0, b * _NTILE + i)),
        ],
        out_shape=[
            jax.ShapeDtypeStruct((_B, _V, _C // 2), jnp.int32),
            jax.ShapeDtypeStruct((_B, _V, _C), jnp.int32),
            jax.ShapeDtypeStruct((_K, _B * _V), jnp.int32),
            jax.ShapeDtypeStruct((_K, _B * _V), jnp.int32),
        ],
    )(tok, mask_i32, g1, b1, wq, bq, wk, bk, wv, bv)


_NIDX = _B * _V * _K
_GCH = 128


def _sc_gather(kvtab, idx):
    info = plsc.get_sparse_core_info()
    nw = info.num_cores * info.num_subcores
    per_w = _NIDX // nw
    n_ch = per_w // _GCH
    mesh = plsc.VectorSubcoreMesh(core_axis_name="c", subcore_axis_name="s")

    @functools.partial(
        pl.kernel,
        mesh=mesh,
        out_type=jax.ShapeDtypeStruct((_NIDX, _C), jnp.int32),
        scratch_types=[
            pltpu.VMEM((_GCH,), jnp.int32),
            pltpu.VMEM((_GCH,), jnp.int32),
            pltpu.VMEM((_GCH, _C), jnp.int32),
            pltpu.VMEM((_GCH, _C), jnp.int32),
            pltpu.SemaphoreType.DMA,
            pltpu.SemaphoreType.DMA,
        ],
    )
    def gath(tab_hbm, idx_hbm, out_hbm,
             idx0, idx1, buf0, buf1, sem0, sem1):
        wid = lax.axis_index("s") * info.num_cores + lax.axis_index("c")
        base = wid * per_w
        idxs = (idx0, idx1)
        bufs = (buf0, buf1)
        sems = (sem0, sem1)

        def start(g, slot):
            off = base + g * _GCH
            pltpu.sync_copy(idx_hbm.at[pl.ds(off, _GCH)], idxs[slot])
            pltpu.async_copy(tab_hbm.at[idxs[slot]], bufs[slot], sems[slot])

        def finish(g, slot):
            pltpu.make_async_copy(
                tab_hbm.at[idxs[slot]], bufs[slot], sems[slot]).wait()
            pltpu.sync_copy(bufs[slot], out_hbm.at[pl.ds(base + g * _GCH, _GCH)])

        start(0, 0)

        def body(j, carry):
            g0 = j * 2
            start(g0 + 1, 1)
            finish(g0, 0)

            @pl.when(g0 + 2 < n_ch)
            def _():
                start(g0 + 2, 0)

            finish(g0 + 1, 1)
            return carry

        lax.fori_loop(0, n_ch // 2, body, 0)

    return gath(kvtab, idx)


def _attn_mlp_body(tok_ref, q_ref, kvnb_ref, val_ref, maskf_ref,
                   g2_ref, b2_ref, wo_ref, bo_ref,
                   w1_ref, bm1_ref, w2_ref, bm2_ref, out_ref):
    hc = _C // 2
    dd = lax.broadcasted_iota(jnp.int32, (_C, _H), 0) // _DH
    hh = lax.broadcasted_iota(jnp.int32, (_C, _H), 1)
    seg = jnp.where(dd == hh, 1.0 / (_DH ** 0.5), 0.0)
    dd2 = lax.broadcasted_iota(jnp.int32, (_H, _C), 1) // _DH
    hh2 = lax.broadcasted_iota(jnp.int32, (_H, _C), 0)
    segT = jnp.where(dd2 == hh2, 1.0, 0.0)

    q_lo, q_hi = _unpack_bf16(q_ref[0])
    s_list = []
    for k in range(_K):
        w = kvnb_ref[k]
        k_lo, k_hi = _unpack_bf16(w[:, :hc])
        prod_k = jnp.concatenate([q_lo * k_lo, q_hi * k_hi], axis=1)
        s_k = jnp.dot(prod_k, seg, preferred_element_type=jnp.float32)
        s_kT = s_k.T
        val_k = val_ref[pl.ds(k, 1), :]
        s_list.append(jnp.where(val_k != 0, s_kT, -1e30))

    rem = list(s_list)
    cum = jnp.zeros((_H, _TILE), jnp.float32)
    thresh = jnp.full((_H, _TILE), -3e38, jnp.float32)
    for _ in range(_TK):
        mt = rem[0]
        for r in rem[1:]:
            mt = jnp.maximum(mt, r)
        eqs = [r == mt for r in rem]
        cnt = eqs[0].astype(jnp.float32)
        for e_ in eqs[1:]:
            cnt = cnt + e_.astype(jnp.float32)
        thresh = jnp.where(cum < float(_TK), mt, thresh)
        cum = cum + cnt
        rem = [jnp.where(e_, -3e38, r) for e_, r in zip(eqs, rem)]

    s2_list = [jnp.where(s >= thresh, s, -1e30) for s in s_list]
    smax = s2_list[0]
    for s in s2_list[1:]:
        smax = jnp.maximum(smax, s)
    e_list = []
    esum = jnp.full((_H, _TILE), 1e-9, jnp.float32)
    for s in s2_list:
        e_ = jnp.exp(s - smax)
        e_ = jnp.where(s > -1e29, e_, 0.0)
        e_list.append(e_)
        esum = esum + e_
    inv = 1.0 / esum

    out_lo = jnp.zeros((_TILE, hc), jnp.float32)
    out_hi = jnp.zeros((_TILE, hc), jnp.float32)
    for k in range(_K):
        attn_k = e_list[k] * inv
        af = lax.dot_general(attn_k, segT,
                             (((0,), (0,)), ((), ())),
                             preferred_element_type=jnp.float32)
        v_lo, v_hi = _unpack_bf16(kvnb_ref[k][:, hc:])
        out_lo = out_lo + af[:, :hc] * v_lo
        out_hi = out_hi + af[:, hc:] * v_hi
    out = jnp.concatenate([out_lo, out_hi], axis=1)
    proj = jnp.dot(out, wo_ref[...], preferred_element_type=jnp.float32) + bo_ref[0]
    proj = proj * maskf_ref[0]
    x1 = proj * 0.5 + tok_ref[0]

    mu = jnp.mean(x1, axis=1, keepdims=True)
    var = jnp.mean((x1 - mu) ** 2, axis=1, keepdims=True)
    xn = (x1 - mu) * lax.rsqrt(var + 1e-5) * g2_ref[0] + b2_ref[0]
    h = jnp.dot(xn, w1_ref[...], preferred_element_type=jnp.float32) + bm1_ref[0]
    h = jax.nn.gelu(h)
    mlp = jnp.dot(h, w2_ref[...], preferred_element_type=jnp.float32) + bm2_ref[0]
    out_ref[0] = mlp * 0.5 + x1


def _run_attn_mlp(tok, q, kvnb, val, maskf, g2, b2, wo, bo, w1, bm1, w2, bm2):
    full = lambda s: pl.BlockSpec(s, lambda b, i: (0,) * len(s))
    grid = (_B, _NTILE)
    return pl.pallas_call(
        _attn_mlp_body,
        grid=grid,
        in_specs=[
            pl.BlockSpec((1, _TILE, _C), lambda b, i: (b, i, 0)),
            pl.BlockSpec((1, _TILE, _C // 2), lambda b, i: (b, i, 0)),
            pl.BlockSpec((_K, _TILE, _C), lambda b, i: (0, b * _NTILE + i, 0)),
            pl.BlockSpec((_K, _TILE), lambda b, i: (0, b * _NTILE + i)),
            pl.BlockSpec((1, _TILE, 1), lambda b, i: (b, i, 0)),
            full((1, _C)), full((1, _C)),
            full((_C, _C)), full((1, _C)),
            full((_C, _MLP)), full((1, _MLP)),
            full((_MLP, _C)), full((1, _C)),
        ],
        out_specs=pl.BlockSpec((1, _TILE, _C), lambda b, i: (b, i, 0)),
        out_shape=jax.ShapeDtypeStruct((_B, _V, _C), jnp.float32),
    )(tok, q, kvnb, val, maskf, g2, b2, wo, bo, w1, bm1, w2, bm2)


def kernel(voxel_tokens, non_empty_mask, g1, b1, g2, b2, Wq, bq, Wk, bk,
           Wv, bv, Wo, bo, W1, bm1, W2, bm2):
    mask_i32 = non_empty_mask.astype(jnp.int32).reshape(_B, 1, _V)
    r2 = lambda a: a.reshape(1, -1)

    q, kv, idx, val = _run_qkv_knn(
        voxel_tokens, mask_i32, r2(g1), r2(b1),
        Wq, r2(bq), Wk, r2(bk), Wv, r2(bv))

    kvnb = _sc_gather(kv.reshape(_B * _V, _C), idx.reshape(_K * _B * _V))

    maskf = non_empty_mask.astype(jnp.float32).reshape(_B, _V, 1)
    return _run_attn_mlp(
        voxel_tokens, q, kvnb.reshape(_K, _B * _V, _C), val, maskf,
        r2(g2), r2(b2), Wo, r2(bo), W1, r2(bm1), W2, r2(bm2))


# speedup vs baseline: 31.5963x; 1.1679x over previous
import functools

import jax
import jax.numpy as jnp
from jax import lax
from jax.experimental import pallas as pl
from jax.experimental.pallas import tpu as pltpu
from jax.experimental.pallas import tpu_sc as plsc

_R = 16
_V = _R ** 3
_B = 2
_C = 384
_H = 8
_DH = _C // _H
_K = 16
_TK = 8
_MLP = 1536
_TILE = 256
_NTILE = _V // _TILE
_BIG_D2 = 676


def _pack_bf16(lo, hi):
    def rnd(x):
        u = lax.bitcast_convert_type(x, jnp.int32)
        return lax.shift_right_logical(
            u + 0x7FFF + (lax.shift_right_logical(u, 16) & 1), 16)
    return rnd(lo) | (rnd(hi) << 16)


def _unpack_bf16(words):
    lo = lax.bitcast_convert_type(words << 16, jnp.float32)
    hi = lax.bitcast_convert_type(words & jnp.int32(-65536), jnp.float32)
    return lo, hi


def _qkv_knn_body(tok_ref, mask_ref, g1_ref, b1_ref, wq_ref, bq_ref,
                  wk_ref, bk_ref, wv_ref, bv_ref,
                  q_ref, kv_ref, idx_ref, val_ref):
    b = pl.program_id(0)
    i = pl.program_id(1)

    x = tok_ref[0]
    mu = jnp.mean(x, axis=1, keepdims=True)
    var = jnp.mean((x - mu) ** 2, axis=1, keepdims=True)
    xn = (x - mu) * lax.rsqrt(var + 1e-5)
    xn = xn * g1_ref[0] + b1_ref[0]

    q = jnp.dot(xn, wq_ref[...], preferred_element_type=jnp.float32) + bq_ref[0]
    k = jnp.dot(xn, wk_ref[...], preferred_element_type=jnp.float32) + bk_ref[0]
    v = jnp.dot(xn, wv_ref[...], preferred_element_type=jnp.float32) + bv_ref[0]
    hc = _C // 2
    q_ref[0] = _pack_bf16(q[:, :hc], q[:, hc:])
    kv_ref[0] = jnp.concatenate(
        [_pack_bf16(k[:, :hc], k[:, hc:]), _pack_bf16(v[:, :hc], v[:, hc:])],
        axis=1)

    rr = lax.broadcasted_iota(jnp.int32, (_TILE, _V), 0) + i * _TILE
    cc = lax.broadcasted_iota(jnp.int32, (_TILE, _V), 1)
    dx = (rr >> 8) - (cc >> 8)
    dy = ((rr >> 4) & 15) - ((cc >> 4) & 15)
    dz = (rr & 15) - (cc & 15)
    dxf = dx.astype(jnp.float32)
    dyf = dy.astype(jnp.float32)
    dzf = dz.astype(jnp.float32)
    d2 = dxf * dxf + dyf * dyf + dzf * dzf
    mrow = mask_ref[0]
    d2 = jnp.where(mrow != 0, d2, float(_BIG_D2))
    keys = d2 * float(_V) + cc.astype(jnp.float32)

    valid_cut = float(_BIG_D2 * _V)
    idx_cols = []
    val_cols = []
    for _ in range(_K):
        m = jnp.min(keys, axis=1, keepdims=True)
        mi = m.astype(jnp.int32)
        idx_cols.append((mi & (_V - 1)) + b * _V)
        val_cols.append((m < valid_cut).astype(jnp.int32))
        keys = jnp.where(keys == m, 3e38, keys)
    idx_ref[...] = jnp.concatenate(idx_cols, axis=1).T
    val_ref[...] = jnp.concatenate(val_cols, axis=1).T


def _run_qkv_knn(tok, mask_i32, g1, b1, wq, bq, wk, bk, wv, bv):
    full = lambda s: pl.BlockSpec(s, lambda b, i: (0,) * len(s))
    grid = (1, _NTILE)
    return pl.pallas_call(
        _qkv_knn_body,
        grid=grid,
        in_specs=[
            pl.BlockSpec((1, _TILE, _C), lambda b, i: (b, i, 0)),
            pl.BlockSpec((1, 1, _V), lambda b, i: (b, 0, 0)),
            full((1, _C)), full((1, _C)),
            full((_C, _C)), full((1, _C)),
            full((_C, _C)), full((1, _C)),
            full((_C, _C)), full((1, _C)),
        ],
        out_specs=[
            pl.BlockSpec((1, _TILE, _C // 2), lambda b, i: (b, i, 0)),
            pl.BlockSpec((1, _TILE, _C), lambda b, i: (b, i, 0)),
            pl.BlockSpec((_K, _TILE), lambda b, i: (0, b * _NTILE + i)),
            pl.BlockSpec((_K, _TILE), lambda b, i: (0, b * _NTILE + i)),
        ],
        out_shape=[
            jax.ShapeDtypeStruct((1, _V, _C // 2), jnp.int32),
            jax.ShapeDtypeStruct((1, _V, _C), jnp.int32),
            jax.ShapeDtypeStruct((_K, _V), jnp.int32),
            jax.ShapeDtypeStruct((_K, _V), jnp.int32),
        ],
    )(tok, mask_i32, g1, b1, wq, bq, wk, bk, wv, bv)


_NIDX = _V * _K
_GCH = 128


def _sc_gather(kvtab, idx):
    info = plsc.get_sparse_core_info()
    nw = info.num_cores * info.num_subcores
    per_w = _NIDX // nw
    n_ch = per_w // _GCH
    mesh = plsc.VectorSubcoreMesh(core_axis_name="c", subcore_axis_name="s")

    @functools.partial(
        pl.kernel,
        mesh=mesh,
        out_type=jax.ShapeDtypeStruct((_NIDX, _C), jnp.int32),
        scratch_types=[
            pltpu.VMEM((_GCH,), jnp.int32),
            pltpu.VMEM((_GCH,), jnp.int32),
            pltpu.VMEM((_GCH, _C), jnp.int32),
            pltpu.VMEM((_GCH, _C), jnp.int32),
            pltpu.SemaphoreType.DMA,
            pltpu.SemaphoreType.DMA,
        ],
    )
    def gath(tab_hbm, idx_hbm, out_hbm,
             idx0, idx1, buf0, buf1, sem0, sem1):
        wid = lax.axis_index("s") * info.num_cores + lax.axis_index("c")
        base = wid * per_w
        idxs = (idx0, idx1)
        bufs = (buf0, buf1)
        sems = (sem0, sem1)

        def start(g, slot):
            off = base + g * _GCH
            pltpu.sync_copy(idx_hbm.at[pl.ds(off, _GCH)], idxs[slot])
            pltpu.async_copy(tab_hbm.at[idxs[slot]], bufs[slot], sems[slot])

        def finish(g, slot):
            pltpu.make_async_copy(
                tab_hbm.at[idxs[slot]], bufs[slot], sems[slot]).wait()
            pltpu.sync_copy(bufs[slot], out_hbm.at[pl.ds(base + g * _GCH, _GCH)])

        start(0, 0)

        def body(j, carry):
            g0 = j * 2
            start(g0 + 1, 1)
            finish(g0, 0)

            @pl.when(g0 + 2 < n_ch)
            def _():
                start(g0 + 2, 0)

            finish(g0 + 1, 1)
            return carry

        lax.fori_loop(0, n_ch // 2, body, 0)

    return gath(kvtab, idx)


def _attn_mlp_body(tok_ref, q_ref, kvnb_ref, val_ref, maskf_ref,
                   g2_ref, b2_ref, wo_ref, bo_ref,
                   w1_ref, bm1_ref, w2_ref, bm2_ref, out_ref):
    hc = _C // 2
    dd = lax.broadcasted_iota(jnp.int32, (_C, _H), 0) // _DH
    hh = lax.broadcasted_iota(jnp.int32, (_C, _H), 1)
    seg = jnp.where(dd == hh, 1.0 / (_DH ** 0.5), 0.0)
    dd2 = lax.broadcasted_iota(jnp.int32, (_H, _C), 1) // _DH
    hh2 = lax.broadcasted_iota(jnp.int32, (_H, _C), 0)
    segT = jnp.where(dd2 == hh2, 1.0, 0.0)

    q_lo, q_hi = _unpack_bf16(q_ref[0])
    s_list = []
    for k in range(_K):
        w = kvnb_ref[k]
        k_lo, k_hi = _unpack_bf16(w[:, :hc])
        prod_k = jnp.concatenate([q_lo * k_lo, q_hi * k_hi], axis=1)
        s_k = jnp.dot(prod_k, seg, preferred_element_type=jnp.float32)
        s_kT = s_k.T
        val_k = val_ref[pl.ds(k, 1), :]
        s_list.append(jnp.where(val_k != 0, s_kT, -1e30))

    rem = list(s_list)
    cum = jnp.zeros((_H, _TILE), jnp.float32)
    thresh = jnp.full((_H, _TILE), -3e38, jnp.float32)
    for _ in range(_TK):
        mt = rem[0]
        for r in rem[1:]:
            mt = jnp.maximum(mt, r)
        eqs = [r == mt for r in rem]
        cnt = eqs[0].astype(jnp.float32)
        for e_ in eqs[1:]:
            cnt = cnt + e_.astype(jnp.float32)
        thresh = jnp.where(cum < float(_TK), mt, thresh)
        cum = cum + cnt
        rem = [jnp.where(e_, -3e38, r) for e_, r in zip(eqs, rem)]

    s2_list = [jnp.where(s >= thresh, s, -1e30) for s in s_list]
    smax = s2_list[0]
    for s in s2_list[1:]:
        smax = jnp.maximum(smax, s)
    e_list = []
    esum = jnp.full((_H, _TILE), 1e-9, jnp.float32)
    for s in s2_list:
        e_ = jnp.exp(s - smax)
        e_ = jnp.where(s > -1e29, e_, 0.0)
        e_list.append(e_)
        esum = esum + e_
    inv = 1.0 / esum

    out_lo = jnp.zeros((_TILE, hc), jnp.float32)
    out_hi = jnp.zeros((_TILE, hc), jnp.float32)
    for k in range(_K):
        attn_k = e_list[k] * inv
        af = lax.dot_general(attn_k, segT,
                             (((0,), (0,)), ((), ())),
                             preferred_element_type=jnp.float32)
        v_lo, v_hi = _unpack_bf16(kvnb_ref[k][:, hc:])
        out_lo = out_lo + af[:, :hc] * v_lo
        out_hi = out_hi + af[:, hc:] * v_hi
    out = jnp.concatenate([out_lo, out_hi], axis=1)
    proj = jnp.dot(out, wo_ref[...], preferred_element_type=jnp.float32) + bo_ref[0]
    proj = proj * maskf_ref[0]
    x1 = proj * 0.5 + tok_ref[0]

    mu = jnp.mean(x1, axis=1, keepdims=True)
    var = jnp.mean((x1 - mu) ** 2, axis=1, keepdims=True)
    xn = (x1 - mu) * lax.rsqrt(var + 1e-5) * g2_ref[0] + b2_ref[0]
    h = jnp.dot(xn, w1_ref[...], preferred_element_type=jnp.float32) + bm1_ref[0]
    h = jax.nn.gelu(h)
    mlp = jnp.dot(h, w2_ref[...], preferred_element_type=jnp.float32) + bm2_ref[0]
    out_ref[0] = mlp * 0.5 + x1


def _run_attn_mlp(tok, q, kvnb, val, maskf, g2, b2, wo, bo, w1, bm1, w2, bm2):
    full = lambda s: pl.BlockSpec(s, lambda b, i: (0,) * len(s))
    grid = (1, _NTILE)
    return pl.pallas_call(
        _attn_mlp_body,
        grid=grid,
        in_specs=[
            pl.BlockSpec((1, _TILE, _C), lambda b, i: (b, i, 0)),
            pl.BlockSpec((1, _TILE, _C // 2), lambda b, i: (b, i, 0)),
            pl.BlockSpec((_K, _TILE, _C), lambda b, i: (0, b * _NTILE + i, 0)),
            pl.BlockSpec((_K, _TILE), lambda b, i: (0, b * _NTILE + i)),
            pl.BlockSpec((1, _TILE, 1), lambda b, i: (b, i, 0)),
            full((1, _C)), full((1, _C)),
            full((_C, _C)), full((1, _C)),
            full((_C, _MLP)), full((1, _MLP)),
            full((_MLP, _C)), full((1, _C)),
        ],
        out_specs=pl.BlockSpec((1, _TILE, _C), lambda b, i: (b, i, 0)),
        out_shape=jax.ShapeDtypeStruct((1, _V, _C), jnp.float32),
    )(tok, q, kvnb, val, maskf, g2, b2, wo, bo, w1, bm1, w2, bm2)


def kernel(voxel_tokens, non_empty_mask, g1, b1, g2, b2, Wq, bq, Wk, bk,
           Wv, bv, Wo, bo, W1, bm1, W2, bm2):
    mask_i32 = non_empty_mask.astype(jnp.int32).reshape(_B, 1, _V)
    maskf_all = non_empty_mask.astype(jnp.float32).reshape(_B, _V, 1)
    r2 = lambda a: a.reshape(1, -1)

    qs, kvnbs, vals, toks, maskfs = [], [], [], [], []
    for b in range(_B):
        tok_b = lax.slice_in_dim(voxel_tokens, b, b + 1, axis=0)
        q, kv, idx, val = _run_qkv_knn(
            tok_b, lax.slice_in_dim(mask_i32, b, b + 1, axis=0),
            r2(g1), r2(b1), Wq, r2(bq), Wk, r2(bk), Wv, r2(bv))
        kvnb = _sc_gather(kv.reshape(_V, _C), idx.reshape(_K * _V))
        qs.append(q)
        kvnbs.append(kvnb)
        vals.append(val)
        toks.append(tok_b)
        maskfs.append(lax.slice_in_dim(maskf_all, b, b + 1, axis=0))

    outs = []
    for b in range(_B):
        outs.append(_run_attn_mlp(
            toks[b], qs[b], kvnbs[b].reshape(_K, _V, _C), vals[b],
            maskfs[b], r2(g2), r2(b2), Wo, r2(bo), W1, r2(bm1),
            W2, r2(bm2)))
    return jnp.concatenate(outs, axis=0)


# speedup vs baseline: 35.7297x; 1.1308x over previous
import functools

import jax
import jax.numpy as jnp
from jax import lax
from jax.experimental import pallas as pl
from jax.experimental.pallas import tpu as pltpu
from jax.experimental.pallas import tpu_sc as plsc

_R = 16
_V = _R ** 3
_B = 2
_C = 384
_H = 8
_DH = _C // _H
_K = 16
_TK = 8
_MLP = 1536
_TILE = 256
_NTILE = _V // _TILE
_BIG_D2 = 676


def _pack_bf16(lo, hi):
    def rnd(x):
        u = lax.bitcast_convert_type(x, jnp.int32)
        return lax.shift_right_logical(
            u + 0x7FFF + (lax.shift_right_logical(u, 16) & 1), 16)
    return rnd(lo) | (rnd(hi) << 16)


def _unpack_bf16(words):
    lo = lax.bitcast_convert_type(words << 16, jnp.float32)
    hi = lax.bitcast_convert_type(words & jnp.int32(-65536), jnp.float32)
    return lo, hi


def _qkv_knn_body(tok_ref, mask_ref, g1_ref, b1_ref, wq_ref, bq_ref,
                  wk_ref, bk_ref, wv_ref, bv_ref,
                  q_ref, kv_ref, idx_ref, val_ref):
    b = pl.program_id(0)
    i = pl.program_id(1)

    x = tok_ref[0]
    mu = jnp.mean(x, axis=1, keepdims=True)
    var = jnp.mean((x - mu) ** 2, axis=1, keepdims=True)
    xn = (x - mu) * lax.rsqrt(var + 1e-5)
    xn = xn * g1_ref[0] + b1_ref[0]

    q = jnp.dot(xn, wq_ref[...], preferred_element_type=jnp.float32) + bq_ref[0]
    k = jnp.dot(xn, wk_ref[...], preferred_element_type=jnp.float32) + bk_ref[0]
    v = jnp.dot(xn, wv_ref[...], preferred_element_type=jnp.float32) + bv_ref[0]
    hc = _C // 2
    q_ref[0] = _pack_bf16(q[:, :hc], q[:, hc:])
    kv_ref[0] = jnp.concatenate(
        [_pack_bf16(k[:, :hc], k[:, hc:]), _pack_bf16(v[:, :hc], v[:, hc:])],
        axis=1)

    ty = lax.broadcasted_iota(jnp.int32, (_TILE, 256), 0)
    cyz = lax.broadcasted_iota(jnp.int32, (_TILE, 256), 1)
    dy = (ty >> 4) - (cyz >> 4)
    dz = (ty & 15) - (cyz & 15)
    a1 = ((dy * dy + dz * dz) * _V + cyz).astype(jnp.float32)
    a1t = jnp.tile(a1, (1, _R))
    cc = lax.broadcasted_iota(jnp.int32, (1, _V), 1)
    cx = cc >> 8
    dx = i - cx
    a2 = (dx * dx * _V + cx * 256).astype(jnp.float32)
    big = (cc + _BIG_D2 * _V).astype(jnp.float32)
    mrow = mask_ref[0]
    keys = jnp.where(mrow != 0, a1t + a2, big)

    def cmpswap(a, b):
        return jnp.minimum(a, b), jnp.maximum(a, b)

    qw = _V // 4
    w0 = keys[:, 0 * qw:1 * qw]
    w1 = keys[:, 1 * qw:2 * qw]
    w2 = keys[:, 2 * qw:3 * qw]
    w3 = keys[:, 3 * qw:4 * qw]
    r0, r1 = cmpswap(w0, w1)
    r2, r3 = cmpswap(w2, w3)
    r0, r2 = cmpswap(r0, r2)
    r1, r3 = cmpswap(r1, r3)
    r1, r2 = cmpswap(r1, r2)

    idx_cols = []
    val_cols = []
    for _ in range(_K):
        m = jnp.min(r0, axis=1, keepdims=True)
        mi = m.astype(jnp.int32)
        idx_cols.append(mi & (_V - 1))
        val_cols.append((mi < _BIG_D2 * _V).astype(jnp.int32))
        eq = r0 == m
        r0 = jnp.where(eq, r1, r0)
        r1 = jnp.where(eq, r2, r1)
        r2 = jnp.where(eq, r3, r2)
        r3 = jnp.where(eq, 3e38, r3)
    idx_ref[...] = jnp.concatenate(idx_cols, axis=1).T
    val_ref[...] = jnp.concatenate(val_cols, axis=1).T


def _run_qkv_knn(tok, mask_i32, g1, b1, wq, bq, wk, bk, wv, bv):
    full = lambda s: pl.BlockSpec(s, lambda b, i: (0,) * len(s))
    grid = (1, _NTILE)
    return pl.pallas_call(
        _qkv_knn_body,
        grid=grid,
        in_specs=[
            pl.BlockSpec((1, _TILE, _C), lambda b, i: (b, i, 0)),
            pl.BlockSpec((1, 1, _V), lambda b, i: (b, 0, 0)),
            full((1, _C)), full((1, _C)),
            full((_C, _C)), full((1, _C)),
            full((_C, _C)), full((1, _C)),
            full((_C, _C)), full((1, _C)),
        ],
        out_specs=[
            pl.BlockSpec((1, _TILE, _C // 2), lambda b, i: (b, i, 0)),
            pl.BlockSpec((1, _TILE, _C), lambda b, i: (b, i, 0)),
            pl.BlockSpec((_K, _TILE), lambda b, i: (0, b * _NTILE + i)),
            pl.BlockSpec((_K, _TILE), lambda b, i: (0, b * _NTILE + i)),
        ],
        out_shape=[
            jax.ShapeDtypeStruct((1, _V, _C // 2), jnp.int32),
            jax.ShapeDtypeStruct((1, _V, _C), jnp.int32),
            jax.ShapeDtypeStruct((_K, _V), jnp.int32),
            jax.ShapeDtypeStruct((_K, _V), jnp.int32),
        ],
    )(tok, mask_i32, g1, b1, wq, bq, wk, bk, wv, bv)


_NIDX = _V * _K
_GCH = 128


def _sc_gather(kvtab, idx):
    info = plsc.get_sparse_core_info()
    nw = info.num_cores * info.num_subcores
    per_w = _NIDX // nw
    n_ch = per_w // _GCH
    mesh = plsc.VectorSubcoreMesh(core_axis_name="c", subcore_axis_name="s")

    @functools.partial(
        pl.kernel,
        mesh=mesh,
        out_type=jax.ShapeDtypeStruct((_NIDX, _C), jnp.int32),
        scratch_types=[
            pltpu.VMEM((_GCH,), jnp.int32),
            pltpu.VMEM((_GCH,), jnp.int32),
            pltpu.VMEM((_GCH, _C), jnp.int32),
            pltpu.VMEM((_GCH, _C), jnp.int32),
            pltpu.SemaphoreType.DMA,
            pltpu.SemaphoreType.DMA,
        ],
    )
    def gath(tab_hbm, idx_hbm, out_hbm,
             idx0, idx1, buf0, buf1, sem0, sem1):
        wid = lax.axis_index("s") * info.num_cores + lax.axis_index("c")
        base = wid * per_w
        idxs = (idx0, idx1)
        bufs = (buf0, buf1)
        sems = (sem0, sem1)

        def start(g, slot):
            off = base + g * _GCH
            pltpu.sync_copy(idx_hbm.at[pl.ds(off, _GCH)], idxs[slot])
            pltpu.async_copy(tab_hbm.at[idxs[slot]], bufs[slot], sems[slot])

        def finish(g, slot):
            pltpu.make_async_copy(
                tab_hbm.at[idxs[slot]], bufs[slot], sems[slot]).wait()
            pltpu.sync_copy(bufs[slot], out_hbm.at[pl.ds(base + g * _GCH, _GCH)])

        start(0, 0)

        def body(j, carry):
            g0 = j * 2
            start(g0 + 1, 1)
            finish(g0, 0)

            @pl.when(g0 + 2 < n_ch)
            def _():
                start(g0 + 2, 0)

            finish(g0 + 1, 1)
            return carry

        lax.fori_loop(0, n_ch // 2, body, 0)

    return gath(kvtab, idx)


def _attn_mlp_body(tok_ref, q_ref, kvnb_ref, val_ref, maskf_ref,
                   g2_ref, b2_ref, wo_ref, bo_ref,
                   w1_ref, bm1_ref, w2_ref, bm2_ref, out_ref):
    hc = _C // 2
    dd = lax.broadcasted_iota(jnp.int32, (_C, _H), 0) // _DH
    hh = lax.broadcasted_iota(jnp.int32, (_C, _H), 1)
    seg = jnp.where(dd == hh, 1.0 / (_DH ** 0.5), 0.0)
    dd2 = lax.broadcasted_iota(jnp.int32, (_H, _C), 1) // _DH
    hh2 = lax.broadcasted_iota(jnp.int32, (_H, _C), 0)
    segT = jnp.where(dd2 == hh2, 1.0, 0.0)

    q_lo, q_hi = _unpack_bf16(q_ref[0])
    s_list = []
    for k in range(_K):
        w = kvnb_ref[k]
        k_lo, k_hi = _unpack_bf16(w[:, :hc])
        prod_k = jnp.concatenate([q_lo * k_lo, q_hi * k_hi], axis=1)
        s_k = jnp.dot(prod_k, seg, preferred_element_type=jnp.float32)
        s_kT = s_k.T
        val_k = val_ref[pl.ds(k, 1), :]
        s_list.append(jnp.where(val_k != 0, s_kT, -1e30))

    rem = list(s_list)
    cum = jnp.zeros((_H, _TILE), jnp.float32)
    thresh = jnp.full((_H, _TILE), -3e38, jnp.float32)
    for _ in range(_TK):
        mt = rem[0]
        for r in rem[1:]:
            mt = jnp.maximum(mt, r)
        eqs = [r == mt for r in rem]
        cnt = eqs[0].astype(jnp.float32)
        for e_ in eqs[1:]:
            cnt = cnt + e_.astype(jnp.float32)
        thresh = jnp.where(cum < float(_TK), mt, thresh)
        cum = cum + cnt
        rem = [jnp.where(e_, -3e38, r) for e_, r in zip(eqs, rem)]

    s2_list = [jnp.where(s >= thresh, s, -1e30) for s in s_list]
    smax = s2_list[0]
    for s in s2_list[1:]:
        smax = jnp.maximum(smax, s)
    e_list = []
    esum = jnp.full((_H, _TILE), 1e-9, jnp.float32)
    for s in s2_list:
        e_ = jnp.exp(s - smax)
        e_ = jnp.where(s > -1e29, e_, 0.0)
        e_list.append(e_)
        esum = esum + e_
    inv = 1.0 / esum

    out_lo = jnp.zeros((_TILE, hc), jnp.float32)
    out_hi = jnp.zeros((_TILE, hc), jnp.float32)
    for k in range(_K):
        attn_k = e_list[k] * inv
        af = lax.dot_general(attn_k, segT,
                             (((0,), (0,)), ((), ())),
                             preferred_element_type=jnp.float32)
        v_lo, v_hi = _unpack_bf16(kvnb_ref[k][:, hc:])
        out_lo = out_lo + af[:, :hc] * v_lo
        out_hi = out_hi + af[:, hc:] * v_hi
    out = jnp.concatenate([out_lo, out_hi], axis=1)
    proj = jnp.dot(out, wo_ref[...], preferred_element_type=jnp.float32) + bo_ref[0]
    proj = proj * maskf_ref[0]
    x1 = proj * 0.5 + tok_ref[0]

    mu = jnp.mean(x1, axis=1, keepdims=True)
    var = jnp.mean((x1 - mu) ** 2, axis=1, keepdims=True)
    xn = (x1 - mu) * lax.rsqrt(var + 1e-5) * g2_ref[0] + b2_ref[0]
    h = jnp.dot(xn, w1_ref[...], preferred_element_type=jnp.float32) + bm1_ref[0]
    h = jax.nn.gelu(h)
    mlp = jnp.dot(h, w2_ref[...], preferred_element_type=jnp.float32) + bm2_ref[0]
    out_ref[0] = mlp * 0.5 + x1


def _run_attn_mlp(tok, q, kvnb, val, maskf, g2, b2, wo, bo, w1, bm1, w2, bm2):
    full = lambda s: pl.BlockSpec(s, lambda b, i: (0,) * len(s))
    grid = (1, _NTILE)
    return pl.pallas_call(
        _attn_mlp_body,
        grid=grid,
        in_specs=[
            pl.BlockSpec((1, _TILE, _C), lambda b, i: (b, i, 0)),
            pl.BlockSpec((1, _TILE, _C // 2), lambda b, i: (b, i, 0)),
            pl.BlockSpec((_K, _TILE, _C), lambda b, i: (0, b * _NTILE + i, 0)),
            pl.BlockSpec((_K, _TILE), lambda b, i: (0, b * _NTILE + i)),
            pl.BlockSpec((1, _TILE, 1), lambda b, i: (b, i, 0)),
            full((1, _C)), full((1, _C)),
            full((_C, _C)), full((1, _C)),
            full((_C, _MLP)), full((1, _MLP)),
            full((_MLP, _C)), full((1, _C)),
        ],
        out_specs=pl.BlockSpec((1, _TILE, _C), lambda b, i: (b, i, 0)),
        out_shape=jax.ShapeDtypeStruct((1, _V, _C), jnp.float32),
    )(tok, q, kvnb, val, maskf, g2, b2, wo, bo, w1, bm1, w2, bm2)


def kernel(voxel_tokens, non_empty_mask, g1, b1, g2, b2, Wq, bq, Wk, bk,
           Wv, bv, Wo, bo, W1, bm1, W2, bm2):
    mask_i32 = non_empty_mask.astype(jnp.int32).reshape(_B, 1, _V)
    maskf_all = non_empty_mask.astype(jnp.float32).reshape(_B, _V, 1)
    r2 = lambda a: a.reshape(1, -1)

    qs, kvnbs, vals, toks, maskfs = [], [], [], [], []
    for b in range(_B):
        tok_b = lax.slice_in_dim(voxel_tokens, b, b + 1, axis=0)
        q, kv, idx, val = _run_qkv_knn(
            tok_b, lax.slice_in_dim(mask_i32, b, b + 1, axis=0),
            r2(g1), r2(b1), Wq, r2(bq), Wk, r2(bk), Wv, r2(bv))
        kvnb = _sc_gather(kv.reshape(_V, _C), idx.reshape(_K * _V))
        qs.append(q)
        kvnbs.append(kvnb)
        vals.append(val)
        toks.append(tok_b)
        maskfs.append(lax.slice_in_dim(maskf_all, b, b + 1, axis=0))

    outs = []
    for b in range(_B):
        outs.append(_run_attn_mlp(
            toks[b], qs[b], kvnbs[b].reshape(_K, _V, _C), vals[b],
            maskfs[b], r2(g2), r2(b2), Wo, r2(bo), W1, r2(bm1),
            W2, r2(bm2)))
    return jnp.concatenate(outs, axis=0)


# speedup vs baseline: 36.6322x; 1.0253x over previous
import functools

import jax
import jax.numpy as jnp
from jax import lax
from jax.experimental import pallas as pl
from jax.experimental.pallas import tpu as pltpu
from jax.experimental.pallas import tpu_sc as plsc

_R = 16
_V = _R ** 3
_B = 2
_C = 384
_H = 8
_DH = _C // _H
_K = 16
_TK = 8
_MLP = 1536
_TILE = 256
_NTILE = _V // _TILE
_BIG_D2 = 676


def _pack_bf16(lo, hi):
    def rnd(x):
        u = lax.bitcast_convert_type(x, jnp.int32)
        return lax.shift_right_logical(
            u + 0x7FFF + (lax.shift_right_logical(u, 16) & 1), 16)
    return rnd(lo) | (rnd(hi) << 16)


def _unpack_bf16(words):
    lo = lax.bitcast_convert_type(words << 16, jnp.float32)
    hi = lax.bitcast_convert_type(words & jnp.int32(-65536), jnp.float32)
    return lo, hi


def _qkv_knn_body(tok_ref, mask_ref, g1_ref, b1_ref, wq_ref, bq_ref,
                  wk_ref, bk_ref, wv_ref, bv_ref,
                  q_ref, kv_ref, idx_ref, val_ref):
    b = pl.program_id(0)
    i = pl.program_id(1)

    x = tok_ref[0]
    mu = jnp.mean(x, axis=1, keepdims=True)
    var = jnp.mean((x - mu) ** 2, axis=1, keepdims=True)
    xn = (x - mu) * lax.rsqrt(var + 1e-5)
    xn = xn * g1_ref[0] + b1_ref[0]

    q = jnp.dot(xn, wq_ref[...], preferred_element_type=jnp.float32) + bq_ref[0]
    k = jnp.dot(xn, wk_ref[...], preferred_element_type=jnp.float32) + bk_ref[0]
    v = jnp.dot(xn, wv_ref[...], preferred_element_type=jnp.float32) + bv_ref[0]
    hc = _C // 2
    q_ref[0] = _pack_bf16(q[:, :hc], q[:, hc:])
    kv_ref[0] = jnp.concatenate(
        [_pack_bf16(k[:, :hc], k[:, hc:]), _pack_bf16(v[:, :hc], v[:, hc:])],
        axis=1)

    ty = lax.broadcasted_iota(jnp.int32, (_TILE, 256), 0)
    cyz = lax.broadcasted_iota(jnp.int32, (_TILE, 256), 1)
    dy = (ty >> 4) - (cyz >> 4)
    dz = (ty & 15) - (cyz & 15)
    a1 = ((dy * dy + dz * dz) * _V + cyz).astype(jnp.float32)
    a1t = jnp.tile(a1, (1, _R))
    cc = lax.broadcasted_iota(jnp.int32, (1, _V), 1)
    cx = cc >> 8
    dx = i - cx
    a2 = (dx * dx * _V + cx * 256).astype(jnp.float32)
    big = (cc + _BIG_D2 * _V).astype(jnp.float32)
    mrow = mask_ref[0]
    keys = jnp.where(mrow != 0, a1t + a2, big)

    def cmpswap(a, b):
        return jnp.minimum(a, b), jnp.maximum(a, b)

    qw = _V // 4
    w0 = keys[:, 0 * qw:1 * qw]
    w1 = keys[:, 1 * qw:2 * qw]
    w2 = keys[:, 2 * qw:3 * qw]
    w3 = keys[:, 3 * qw:4 * qw]
    r0, r1 = cmpswap(w0, w1)
    r2, r3 = cmpswap(w2, w3)
    r0, r2 = cmpswap(r0, r2)
    r1, r3 = cmpswap(r1, r3)
    r1, r2 = cmpswap(r1, r2)

    idx_cols = []
    val_cols = []
    for _ in range(_K):
        m = jnp.min(r0, axis=1, keepdims=True)
        mi = m.astype(jnp.int32)
        idx_cols.append(mi & (_V - 1))
        val_cols.append((mi < _BIG_D2 * _V).astype(jnp.int32))
        eq = r0 == m
        r0 = jnp.where(eq, r1, r0)
        r1 = jnp.where(eq, r2, r1)
        r2 = jnp.where(eq, r3, r2)
        r3 = jnp.where(eq, 3e38, r3)
    idx_ref[...] = jnp.concatenate(idx_cols, axis=1).T
    val_ref[...] = jnp.concatenate(val_cols, axis=1).T


def _run_qkv_knn(tok, mask_i32, g1, b1, wq, bq, wk, bk, wv, bv):
    full = lambda s: pl.BlockSpec(s, lambda b, i: (0,) * len(s))
    grid = (1, _NTILE)
    return pl.pallas_call(
        _qkv_knn_body,
        grid=grid,
        in_specs=[
            pl.BlockSpec((1, _TILE, _C), lambda b, i: (b, i, 0)),
            pl.BlockSpec((1, 1, _V), lambda b, i: (b, 0, 0)),
            full((1, _C)), full((1, _C)),
            full((_C, _C)), full((1, _C)),
            full((_C, _C)), full((1, _C)),
            full((_C, _C)), full((1, _C)),
        ],
        out_specs=[
            pl.BlockSpec((1, _TILE, _C // 2), lambda b, i: (b, i, 0)),
            pl.BlockSpec((1, _TILE, _C), lambda b, i: (b, i, 0)),
            pl.BlockSpec((_K, _TILE), lambda b, i: (0, b * _NTILE + i)),
            pl.BlockSpec((_K, _TILE), lambda b, i: (0, b * _NTILE + i)),
        ],
        out_shape=[
            jax.ShapeDtypeStruct((1, _V, _C // 2), jnp.int32),
            jax.ShapeDtypeStruct((1, _V, _C), jnp.int32),
            jax.ShapeDtypeStruct((_K, _V), jnp.int32),
            jax.ShapeDtypeStruct((_K, _V), jnp.int32),
        ],
    )(tok, mask_i32, g1, b1, wq, bq, wk, bk, wv, bv)


_NIDX = _V * _K
_GCH = 128


def _sc_gather(kvtab, idx):
    info = plsc.get_sparse_core_info()
    nw = info.num_cores * info.num_subcores
    per_w = _NIDX // nw
    n_ch = per_w // _GCH
    mesh = plsc.VectorSubcoreMesh(core_axis_name="c", subcore_axis_name="s")

    @functools.partial(
        pl.kernel,
        mesh=mesh,
        out_type=jax.ShapeDtypeStruct((_NIDX, _C), jnp.int32),
        scratch_types=[
            pltpu.VMEM((_GCH,), jnp.int32),
            pltpu.VMEM((_GCH,), jnp.int32),
            pltpu.VMEM((_GCH, _C), jnp.int32),
            pltpu.VMEM((_GCH, _C), jnp.int32),
            pltpu.SemaphoreType.DMA,
            pltpu.SemaphoreType.DMA,
        ],
    )
    def gath(tab_hbm, idx_hbm, out_hbm,
             idx0, idx1, buf0, buf1, sem0, sem1):
        wid = lax.axis_index("s") * info.num_cores + lax.axis_index("c")
        base = wid * per_w
        idxs = (idx0, idx1)
        bufs = (buf0, buf1)
        sems = (sem0, sem1)

        def start(g, slot):
            off = base + g * _GCH
            pltpu.sync_copy(idx_hbm.at[pl.ds(off, _GCH)], idxs[slot])
            pltpu.async_copy(tab_hbm.at[idxs[slot]], bufs[slot], sems[slot])

        def finish(g, slot):
            pltpu.make_async_copy(
                tab_hbm.at[idxs[slot]], bufs[slot], sems[slot]).wait()
            pltpu.sync_copy(bufs[slot], out_hbm.at[pl.ds(base + g * _GCH, _GCH)])

        start(0, 0)

        def body(j, carry):
            g0 = j * 2
            start(g0 + 1, 1)
            finish(g0, 0)

            @pl.when(g0 + 2 < n_ch)
            def _():
                start(g0 + 2, 0)

            finish(g0 + 1, 1)
            return carry

        lax.fori_loop(0, n_ch // 2, body, 0)

    return gath(kvtab, idx)


def _attn_mlp_body(tok_ref, q_ref, kvnb_ref, val_ref, maskf_ref,
                   g2_ref, b2_ref, wo_ref, bo_ref,
                   w1_ref, bm1_ref, w2_ref, bm2_ref, out_ref):
    hc = _C // 2
    dd = lax.broadcasted_iota(jnp.int32, (_C, _H), 0) // _DH
    hh = lax.broadcasted_iota(jnp.int32, (_C, _H), 1)
    seg = jnp.where(dd == hh, 1.0 / (_DH ** 0.5), 0.0)
    dd2 = lax.broadcasted_iota(jnp.int32, (_H, _C), 1) // _DH
    hh2 = lax.broadcasted_iota(jnp.int32, (_H, _C), 0)
    segT = jnp.where(dd2 == hh2, 1.0, 0.0)

    q_lo, q_hi = _unpack_bf16(q_ref[0])
    s_list = []
    seg_lo = seg[:hc]
    seg_hi = seg[hc:]
    for k in range(_K):
        w = kvnb_ref[k]
        k_lo, k_hi = _unpack_bf16(w[:, :hc])
        s_k = (jnp.dot(q_lo * k_lo, seg_lo, preferred_element_type=jnp.float32)
               + jnp.dot(q_hi * k_hi, seg_hi, preferred_element_type=jnp.float32))
        s_kT = s_k.T
        val_k = val_ref[pl.ds(k, 1), :]
        s_list.append(jnp.where(val_k != 0, s_kT, -1e30))

    rem = list(s_list)
    cum = jnp.zeros((_H, _TILE), jnp.float32)
    thresh = jnp.full((_H, _TILE), -3e38, jnp.float32)
    for _ in range(_TK):
        mt = rem[0]
        for r in rem[1:]:
            mt = jnp.maximum(mt, r)
        eqs = [r == mt for r in rem]
        cnt = eqs[0].astype(jnp.float32)
        for e_ in eqs[1:]:
            cnt = cnt + e_.astype(jnp.float32)
        thresh = jnp.where(cum < float(_TK), mt, thresh)
        cum = cum + cnt
        rem = [jnp.where(e_, -3e38, r) for e_, r in zip(eqs, rem)]

    s2_list = [jnp.where(s >= thresh, s, -1e30) for s in s_list]
    smax = s2_list[0]
    for s in s2_list[1:]:
        smax = jnp.maximum(smax, s)
    e_list = []
    esum = jnp.full((_H, _TILE), 1e-9, jnp.float32)
    for s in s2_list:
        e_ = jnp.exp(s - smax)
        e_ = jnp.where(s > -1e29, e_, 0.0)
        e_list.append(e_)
        esum = esum + e_
    inv = 1.0 / esum

    out_lo = jnp.zeros((_TILE, hc), jnp.float32)
    out_hi = jnp.zeros((_TILE, hc), jnp.float32)
    for k in range(_K):
        attn_k = e_list[k] * inv
        af = lax.dot_general(attn_k, segT,
                             (((0,), (0,)), ((), ())),
                             preferred_element_type=jnp.float32)
        v_lo, v_hi = _unpack_bf16(kvnb_ref[k][:, hc:])
        out_lo = out_lo + af[:, :hc] * v_lo
        out_hi = out_hi + af[:, hc:] * v_hi
    out = jnp.concatenate([out_lo, out_hi], axis=1)
    proj = jnp.dot(out, wo_ref[...], preferred_element_type=jnp.float32) + bo_ref[0]
    proj = proj * maskf_ref[0]
    x1 = proj * 0.5 + tok_ref[0]

    mu = jnp.mean(x1, axis=1, keepdims=True)
    var = jnp.mean((x1 - mu) ** 2, axis=1, keepdims=True)
    xn = (x1 - mu) * lax.rsqrt(var + 1e-5) * g2_ref[0] + b2_ref[0]
    h = jnp.dot(xn, w1_ref[...], preferred_element_type=jnp.float32) + bm1_ref[0]
    h = jax.nn.gelu(h)
    mlp = jnp.dot(h, w2_ref[...], preferred_element_type=jnp.float32) + bm2_ref[0]
    out_ref[0] = mlp * 0.5 + x1


def _run_attn_mlp(tok, q, kvnb, val, maskf, g2, b2, wo, bo, w1, bm1, w2, bm2):
    full = lambda s: pl.BlockSpec(s, lambda b, i: (0,) * len(s))
    grid = (1, _NTILE)
    return pl.pallas_call(
        _attn_mlp_body,
        grid=grid,
        in_specs=[
            pl.BlockSpec((1, _TILE, _C), lambda b, i: (b, i, 0)),
            pl.BlockSpec((1, _TILE, _C // 2), lambda b, i: (b, i, 0)),
            pl.BlockSpec((_K, _TILE, _C), lambda b, i: (0, b * _NTILE + i, 0)),
            pl.BlockSpec((_K, _TILE), lambda b, i: (0, b * _NTILE + i)),
            pl.BlockSpec((1, _TILE, 1), lambda b, i: (b, i, 0)),
            full((1, _C)), full((1, _C)),
            full((_C, _C)), full((1, _C)),
            full((_C, _MLP)), full((1, _MLP)),
            full((_MLP, _C)), full((1, _C)),
        ],
        out_specs=pl.BlockSpec((1, _TILE, _C), lambda b, i: (b, i, 0)),
        out_shape=jax.ShapeDtypeStruct((1, _V, _C), jnp.float32),
    )(tok, q, kvnb, val, maskf, g2, b2, wo, bo, w1, bm1, w2, bm2)


def kernel(voxel_tokens, non_empty_mask, g1, b1, g2, b2, Wq, bq, Wk, bk,
           Wv, bv, Wo, bo, W1, bm1, W2, bm2):
    mask_i32 = non_empty_mask.astype(jnp.int32).reshape(_B, 1, _V)
    maskf_all = non_empty_mask.astype(jnp.float32).reshape(_B, _V, 1)
    r2 = lambda a: a.reshape(1, -1)

    qs, kvnbs, vals, toks, maskfs = [], [], [], [], []
    for b in range(_B):
        tok_b = lax.slice_in_dim(voxel_tokens, b, b + 1, axis=0)
        q, kv, idx, val = _run_qkv_knn(
            tok_b, lax.slice_in_dim(mask_i32, b, b + 1, axis=0),
            r2(g1), r2(b1), Wq, r2(bq), Wk, r2(bk), Wv, r2(bv))
        kvnb = _sc_gather(kv.reshape(_V, _C), idx.reshape(_K * _V))
        qs.append(q)
        kvnbs.append(kvnb)
        vals.append(val)
        toks.append(tok_b)
        maskfs.append(lax.slice_in_dim(maskf_all, b, b + 1, axis=0))

    outs = []
    for b in range(_B):
        outs.append(_run_attn_mlp(
            toks[b], qs[b], kvnbs[b].reshape(_K, _V, _C), vals[b],
            maskfs[b], r2(g2), r2(b2), Wo, r2(bo), W1, r2(bm1),
            W2, r2(bm2)))
    return jnp.concatenate(outs, axis=0)


# speedup vs baseline: 36.6357x; 1.0001x over previous
import functools

import jax
import jax.numpy as jnp
from jax import lax
from jax.experimental import pallas as pl
from jax.experimental.pallas import tpu as pltpu
from jax.experimental.pallas import tpu_sc as plsc

_R = 16
_V = _R ** 3
_B = 2
_C = 384
_H = 8
_DH = _C // _H
_K = 16
_TK = 8
_MLP = 1536
_TILE = 256
_NTILE = _V // _TILE
_BIG_D2 = 676


def _pack_bf16(lo, hi):
    def rnd(x):
        u = lax.bitcast_convert_type(x, jnp.int32)
        return lax.shift_right_logical(
            u + 0x7FFF + (lax.shift_right_logical(u, 16) & 1), 16)
    return rnd(lo) | (rnd(hi) << 16)


def _unpack_bf16(words):
    lo = lax.bitcast_convert_type(words << 16, jnp.float32)
    hi = lax.bitcast_convert_type(words & jnp.int32(-65536), jnp.float32)
    return lo, hi


def _qkv_knn_body(tok_ref, mask_ref, g1_ref, b1_ref, wq_ref, bq_ref,
                  wk_ref, bk_ref, wv_ref, bv_ref,
                  q_ref, kv_ref, idx_ref, val_ref):
    i = pl.program_id(1)

    x = tok_ref[0]
    mu = jnp.mean(x, axis=1, keepdims=True)
    var = jnp.mean((x - mu) ** 2, axis=1, keepdims=True)
    xn = (x - mu) * lax.rsqrt(var + 1e-5)
    xn = xn * g1_ref[0] + b1_ref[0]

    q = jnp.dot(xn, wq_ref[...], preferred_element_type=jnp.float32) + bq_ref[0]
    k = jnp.dot(xn, wk_ref[...], preferred_element_type=jnp.float32) + bk_ref[0]
    v = jnp.dot(xn, wv_ref[...], preferred_element_type=jnp.float32) + bv_ref[0]
    hc = _C // 2
    q_ref[0] = _pack_bf16(q[:, :hc], q[:, hc:])
    kv_ref[0] = jnp.concatenate(
        [_pack_bf16(k[:, :hc], k[:, hc:]), _pack_bf16(v[:, :hc], v[:, hc:])],
        axis=1)

    ty = lax.broadcasted_iota(jnp.int32, (_TILE, 256), 0)
    cyz = lax.broadcasted_iota(jnp.int32, (_TILE, 256), 1)
    dy = (ty >> 4) - (cyz >> 4)
    dz = (ty & 15) - (cyz & 15)
    a1 = ((dy * dy + dz * dz) * _V + cyz).astype(jnp.float32)
    a1t = jnp.tile(a1, (1, _R))
    cc = lax.broadcasted_iota(jnp.int32, (1, _V), 1)
    cx = cc >> 8
    dx = i - cx
    a2 = (dx * dx * _V + cx * 256).astype(jnp.float32)
    big = (cc + _BIG_D2 * _V).astype(jnp.float32)
    mrow = mask_ref[0]
    keys = jnp.where(mrow != 0, a1t + a2, big)

    def cmpswap(a, b):
        return jnp.minimum(a, b), jnp.maximum(a, b)

    qw = _V // 4
    w0 = keys[:, 0 * qw:1 * qw]
    w1 = keys[:, 1 * qw:2 * qw]
    w2 = keys[:, 2 * qw:3 * qw]
    w3 = keys[:, 3 * qw:4 * qw]
    r0, r1 = cmpswap(w0, w1)
    r2, r3 = cmpswap(w2, w3)
    r0, r2 = cmpswap(r0, r2)
    r1, r3 = cmpswap(r1, r3)
    r1, r2 = cmpswap(r1, r2)

    idx_cols = []
    val_cols = []
    for _ in range(_K):
        m = jnp.min(r0, axis=1, keepdims=True)
        mi = m.astype(jnp.int32)
        idx_cols.append(mi & (_V - 1))
        val_cols.append((mi < _BIG_D2 * _V).astype(jnp.int32))
        eq = r0 == m
        r0 = jnp.where(eq, r1, r0)
        r1 = jnp.where(eq, r2, r1)
        r2 = jnp.where(eq, r3, r2)
        r3 = jnp.where(eq, 3e38, r3)
    idx_ref[...] = jnp.concatenate(idx_cols, axis=1).T
    val_ref[...] = jnp.concatenate(val_cols, axis=1).T


def _run_qkv_knn(tok, mask_i32, g1, b1, wq, bq, wk, bk, wv, bv):
    full = lambda s: pl.BlockSpec(s, lambda b, i: (0,) * len(s))
    grid = (1, _NTILE)
    return pl.pallas_call(
        _qkv_knn_body,
        grid=grid,
        in_specs=[
            pl.BlockSpec((1, _TILE, _C), lambda b, i: (b, i, 0)),
            pl.BlockSpec((1, 1, _V), lambda b, i: (b, 0, 0)),
            full((1, _C)), full((1, _C)),
            full((_C, _C)), full((1, _C)),
            full((_C, _C)), full((1, _C)),
            full((_C, _C)), full((1, _C)),
        ],
        out_specs=[
            pl.BlockSpec((1, _TILE, _C // 2), lambda b, i: (b, i, 0)),
            pl.BlockSpec((1, _TILE, _C), lambda b, i: (b, i, 0)),
            pl.BlockSpec((_K, _TILE), lambda b, i: (0, b * _NTILE + i)),
            pl.BlockSpec((_K, _TILE), lambda b, i: (0, b * _NTILE + i)),
        ],
        out_shape=[
            jax.ShapeDtypeStruct((1, _V, _C // 2), jnp.int32),
            jax.ShapeDtypeStruct((1, _V, _C), jnp.int32),
            jax.ShapeDtypeStruct((_K, _V), jnp.int32),
            jax.ShapeDtypeStruct((_K, _V), jnp.int32),
        ],
    )(tok, mask_i32, g1, b1, wq, bq, wk, bk, wv, bv)


_NIDX = _V * _K
_GCH = 128


def _sc_gather(kvtab, idx):
    info = plsc.get_sparse_core_info()
    nw = info.num_cores * info.num_subcores
    per_w = _NIDX // nw
    n_ch = per_w // _GCH
    mesh = plsc.VectorSubcoreMesh(core_axis_name="c", subcore_axis_name="s")

    @functools.partial(
        pl.kernel,
        mesh=mesh,
        out_type=jax.ShapeDtypeStruct((_NIDX, _C), jnp.int32),
        scratch_types=[
            pltpu.VMEM((_GCH,), jnp.int32),
            pltpu.VMEM((_GCH,), jnp.int32),
            pltpu.VMEM((_GCH, _C), jnp.int32),
            pltpu.VMEM((_GCH, _C), jnp.int32),
            pltpu.SemaphoreType.DMA,
            pltpu.SemaphoreType.DMA,
        ],
    )
    def gath(tab_hbm, idx_hbm, out_hbm,
             idx0, idx1, buf0, buf1, sem0, sem1):
        wid = lax.axis_index("s") * info.num_cores + lax.axis_index("c")
        base = wid * per_w
        idxs = (idx0, idx1)
        bufs = (buf0, buf1)
        sems = (sem0, sem1)

        def start(g, slot):
            off = base + g * _GCH
            pltpu.sync_copy(idx_hbm.at[pl.ds(off, _GCH)], idxs[slot])
            pltpu.async_copy(tab_hbm.at[idxs[slot]], bufs[slot], sems[slot])

        def finish(g, slot):
            pltpu.make_async_copy(
                tab_hbm.at[idxs[slot]], bufs[slot], sems[slot]).wait()
            pltpu.sync_copy(bufs[slot], out_hbm.at[pl.ds(base + g * _GCH, _GCH)])

        start(0, 0)

        def body(j, carry):
            g0 = j * 2
            start(g0 + 1, 1)
            finish(g0, 0)

            @pl.when(g0 + 2 < n_ch)
            def _():
                start(g0 + 2, 0)

            finish(g0 + 1, 1)
            return carry

        lax.fori_loop(0, n_ch // 2, body, 0)

    return gath(kvtab, idx)


def _attn_mlp_body(tok_ref, q_ref, kvnb_ref, val_ref, maskf_ref,
                   g2_ref, b2_ref, wo_ref, bo_ref,
                   w1_ref, bm1_ref, w2_ref, bm2_ref, out_ref):
    hc = _C // 2
    dd = lax.broadcasted_iota(jnp.int32, (_C, _H), 0) // _DH
    hh = lax.broadcasted_iota(jnp.int32, (_C, _H), 1)
    seg = jnp.where(dd == hh, 1.0 / (_DH ** 0.5), 0.0)
    dd2 = lax.broadcasted_iota(jnp.int32, (_H, _C), 1) // _DH
    hh2 = lax.broadcasted_iota(jnp.int32, (_H, _C), 0)
    segT = jnp.where(dd2 == hh2, 1.0, 0.0)

    q_lo, q_hi = _unpack_bf16(q_ref[0])
    s_list = []
    seg_lo = seg[:hc]
    seg_hi = seg[hc:]
    for k in range(_K):
        w = kvnb_ref[k]
        k_lo, k_hi = _unpack_bf16(w[:, :hc])
        s_k = (jnp.dot(q_lo * k_lo, seg_lo, preferred_element_type=jnp.float32)
               + jnp.dot(q_hi * k_hi, seg_hi, preferred_element_type=jnp.float32))
        s_kT = s_k.T
        val_k = val_ref[pl.ds(k, 1), :]
        s_list.append(jnp.where(val_k != 0, s_kT, -1e30))

    rem = list(s_list)
    cum = jnp.zeros((_H, _TILE), jnp.float32)
    thresh = jnp.full((_H, _TILE), -3e38, jnp.float32)
    for _ in range(_TK):
        mt = rem[0]
        for r in rem[1:]:
            mt = jnp.maximum(mt, r)
        eqs = [r == mt for r in rem]
        cnt = eqs[0].astype(jnp.float32)
        for e_ in eqs[1:]:
            cnt = cnt + e_.astype(jnp.float32)
        thresh = jnp.where(cum < float(_TK), mt, thresh)
        cum = cum + cnt
        rem = [jnp.where(e_, -3e38, r) for e_, r in zip(eqs, rem)]

    s2_list = [jnp.where(s >= thresh, s, -1e30) for s in s_list]
    smax = s2_list[0]
    for s in s2_list[1:]:
        smax = jnp.maximum(smax, s)
    e_list = []
    esum = jnp.full((_H, _TILE), 1e-9, jnp.float32)
    for s in s2_list:
        e_ = jnp.exp(s - smax)
        e_ = jnp.where(s > -1e29, e_, 0.0)
        e_list.append(e_)
        esum = esum + e_
    inv = 1.0 / esum

    out_lo = jnp.zeros((_TILE, hc), jnp.float32)
    out_hi = jnp.zeros((_TILE, hc), jnp.float32)
    for k in range(_K):
        attn_k = e_list[k] * inv
        af = lax.dot_general(attn_k, segT,
                             (((0,), (0,)), ((), ())),
                             preferred_element_type=jnp.float32)
        v_lo, v_hi = _unpack_bf16(kvnb_ref[k][:, hc:])
        out_lo = out_lo + af[:, :hc] * v_lo
        out_hi = out_hi + af[:, hc:] * v_hi
    out = jnp.concatenate([out_lo, out_hi], axis=1)
    proj = jnp.dot(out, wo_ref[...], preferred_element_type=jnp.float32) + bo_ref[0]
    proj = proj * maskf_ref[0]
    x1 = proj * 0.5 + tok_ref[0]

    mu = jnp.mean(x1, axis=1, keepdims=True)
    var = jnp.mean((x1 - mu) ** 2, axis=1, keepdims=True)
    xn = (x1 - mu) * lax.rsqrt(var + 1e-5) * g2_ref[0] + b2_ref[0]
    h = jnp.dot(xn, w1_ref[...], preferred_element_type=jnp.float32) + bm1_ref[0]
    h = jax.nn.gelu(h)
    mlp = jnp.dot(h, w2_ref[...], preferred_element_type=jnp.float32) + bm2_ref[0]
    out_ref[0] = mlp * 0.5 + x1


def _run_attn_mlp(tok, q, kvnb, val, maskf, g2, b2, wo, bo, w1, bm1, w2, bm2):
    full = lambda s: pl.BlockSpec(s, lambda b, i: (0,) * len(s))
    grid = (1, _NTILE)
    return pl.pallas_call(
        _attn_mlp_body,
        grid=grid,
        in_specs=[
            pl.BlockSpec((1, _TILE, _C), lambda b, i: (b, i, 0)),
            pl.BlockSpec((1, _TILE, _C // 2), lambda b, i: (b, i, 0)),
            pl.BlockSpec((_K, _TILE, _C), lambda b, i: (0, b * _NTILE + i, 0)),
            pl.BlockSpec((_K, _TILE), lambda b, i: (0, b * _NTILE + i)),
            pl.BlockSpec((1, _TILE, 1), lambda b, i: (b, i, 0)),
            full((1, _C)), full((1, _C)),
            full((_C, _C)), full((1, _C)),
            full((_C, _MLP)), full((1, _MLP)),
            full((_MLP, _C)), full((1, _C)),
        ],
        out_specs=pl.BlockSpec((1, _TILE, _C), lambda b, i: (b, i, 0)),
        out_shape=jax.ShapeDtypeStruct((1, _V, _C), jnp.float32),
    )(tok, q, kvnb, val, maskf, g2, b2, wo, bo, w1, bm1, w2, bm2)


def kernel(voxel_tokens, non_empty_mask, g1, b1, g2, b2, Wq, bq, Wk, bk,
           Wv, bv, Wo, bo, W1, bm1, W2, bm2):
    mask_i32 = non_empty_mask.astype(jnp.int32).reshape(_B, 1, _V)
    maskf_all = non_empty_mask.astype(jnp.float32).reshape(_B, _V, 1)
    r2 = lambda a: a.reshape(1, -1)

    qs, kvnbs, vals, toks, maskfs = [], [], [], [], []
    for b in range(_B):
        tok_b = lax.slice_in_dim(voxel_tokens, b, b + 1, axis=0)
        q, kv, idx, val = _run_qkv_knn(
            tok_b, lax.slice_in_dim(mask_i32, b, b + 1, axis=0),
            r2(g1), r2(b1), Wq, r2(bq), Wk, r2(bk), Wv, r2(bv))
        kvnb = _sc_gather(kv.reshape(_V, _C), idx.reshape(_K * _V))
        qs.append(q)
        kvnbs.append(kvnb)
        vals.append(val)
        toks.append(tok_b)
        maskfs.append(lax.slice_in_dim(maskf_all, b, b + 1, axis=0))

    outs = []
    for b in range(_B):
        outs.append(_run_attn_mlp(
            toks[b], qs[b], kvnbs[b].reshape(_K, _V, _C), vals[b],
            maskfs[b], r2(g2), r2(b2), Wo, r2(bo), W1, r2(bm1),
            W2, r2(bm2)))
    return jnp.concatenate(outs, axis=0)
